# baseline (speedup 1.0000x reference)
import functools

import numpy as np
import jax
import jax.numpy as jnp
from jax import lax
from jax.experimental import pallas as pl
from jax.experimental.pallas import tpu as pltpu
from jax.experimental.pallas import tpu_sc as plsc

_P = 7
_M = 31
_NR = 300
_NCH = 512
_CC = 32
_NCK = _NCH // _CC


def _weight_table_np() -> np.ndarray:
    eps = np.float32(np.finfo(np.float32).eps)
    tab = np.zeros((32, _M, _P), np.float32)
    for s in range(1, 32):
        in_size = np.float32(s)
        inv_scale = in_size / np.float32(_P)
        kernel_scale = np.maximum(inv_scale, np.float32(1.0))
        sample_f = ((np.arange(_P, dtype=np.float32) + np.float32(0.5))
                    * inv_scale - np.float32(0.5))
        idx = np.arange(_M, dtype=np.float32)
        xk = np.abs(sample_f[None, :] - idx[:, None]) / kernel_scale
        w = np.maximum(np.float32(0.0), np.float32(1.0) - np.abs(xk))
        w = np.where((idx < in_size)[:, None], w, np.float32(0.0))
        total = np.sum(w, axis=0, keepdims=True)
        w = np.where(np.abs(total) > np.float32(1000.0) * eps,
                     w / np.where(total != 0, total, np.float32(1.0)),
                     np.float32(0.0))
        valid = (sample_f >= np.float32(-0.5)) & (sample_f <= in_size - np.float32(0.5))
        tab[s] = (w * valid[None, :]).astype(np.float32)
    return np.ascontiguousarray(
        np.broadcast_to(tab[:, :, :, None], (32, _M, _P, 16)).astype(np.float32))


_WTAB = _weight_table_np()


def _bounds_table_np() -> np.ndarray:
    tab = _WTAB[:, :, :, 0]
    bnd = np.zeros((32, 16), np.int32)
    for s in range(1, 32):
        for q in range(_P):
            nz = np.nonzero(tab[s, :, q])[0]
            bnd[s, q] = nz.min()
            bnd[s, 8 + q] = nz.max() + 1
    return bnd


_WBND = _bounds_table_np()


def _roi_body(fm_hbm, rois_hbm, wtab_hbm, wbnd_hbm, out_hbm,
              coords_v, wh_v, ww_v, bnd_v, crop_v, tmp_v, outb_v, cnt_s,
              dsem0, dsem1, osem0, osem1):
    sid = lax.axis_index("s")
    cid = lax.axis_index("c")
    pltpu.sync_copy(rois_hbm, coords_v)
    pltpu.sync_copy(wbnd_hbm, bnd_v)
    half = _NR // 2

    @pl.when(sid == 0)
    def _():
        cnt_s[0] = 0

    plsc.subcore_barrier()

    zero = jnp.zeros((16,), jnp.float32)

    _BUCKETS = (8, 16, 24, 31)

    def crop_rows(r_coords, ci, buf, dsem):
        x, y, h, wb_idx = r_coords
        c0 = ci * _CC
        for bj, wb in enumerate(_BUCKETS):
            @pl.when(wb_idx == bj)
            def _(wb=wb):
                def issue(i, carry):
                    pltpu.async_copy(
                        fm_hbm.at[pl.ds(y + i, 1), pl.ds(x, wb), pl.ds(c0, _CC)],
                        crop_v.at[buf, pl.ds(i, 1), pl.ds(0, wb)], dsem)
                    return carry

                lax.fori_loop(0, h, issue, 0)

    def drain_rows(r_coords, ci, buf, dsem):
        x, y, h, wb_idx = r_coords
        c0 = ci * _CC
        for bj, wb in enumerate(_BUCKETS):
            @pl.when(wb_idx == bj)
            def _(wb=wb):
                def drain(i, carry):
                    pltpu.make_async_copy(
                        fm_hbm.at[pl.ds(y + i, 1), pl.ds(x, wb), pl.ds(c0, _CC)],
                        crop_v.at[buf, pl.ds(i, 1), pl.ds(0, wb)], dsem).wait()
                    return carry

                lax.fori_loop(0, h, drain, 0)

    def compute_chunk(r_bounds, r, ci, buf, obuf):
        h, wlo, whi, hlo, hhi = r_bounds
        c0 = ci * _CC

        def quad_a(ib, carry3):
            i0 = 4 * ib
            for q in range(_P):
                def col_a(j, accs, i0=i0, q=q):
                    wwv = ww_v[0, j, q, :]
                    new = []
                    for rr in range(4):
                        p0 = crop_v[buf, i0 + rr, j, pl.ds(0, 16)]
                        p1 = crop_v[buf, i0 + rr, j, pl.ds(16, 16)]
                        new.append(accs[2 * rr] + p0 * wwv)
                        new.append(accs[2 * rr + 1] + p1 * wwv)
                    return tuple(new)

                accs = lax.fori_loop(wlo[q], whi[q], col_a, (zero,) * 8)
                for rr in range(4):
                    tmp_v[i0 + rr, q, pl.ds(0, 16)] = accs[2 * rr]
                    tmp_v[i0 + rr, q, pl.ds(16, 16)] = accs[2 * rr + 1]
            return carry3

        lax.fori_loop(0, (h + 3) // 4, quad_a, 0)

        for p in range(_P):
            def row_b(i, accs, p=p):
                whv = wh_v[0, i, p, :]
                new = []
                for q in range(_P):
                    t0 = tmp_v[i, q, pl.ds(0, 16)]
                    t1 = tmp_v[i, q, pl.ds(16, 16)]
                    new.append(accs[2 * q] + whv * t0)
                    new.append(accs[2 * q + 1] + whv * t1)
                return tuple(new)

            accs = lax.fori_loop(hlo[p], hhi[p], row_b, (zero,) * (2 * _P))
            for q in range(_P):
                outb_v[obuf, p, q, pl.ds(0, 16)] = accs[2 * q]
                outb_v[obuf, p, q, pl.ds(16, 16)] = accs[2 * q + 1]

        pltpu.async_copy(outb_v.at[obuf],
                         out_hbm.at[r, :, :, pl.ds(c0, _CC)],
                         osem0 if obuf == 0 else osem1)

    def drain_out(r, ci, obuf):
        c0 = ci * _CC
        pltpu.make_async_copy(outb_v.at[obuf],
                              out_hbm.at[r, :, :, pl.ds(c0, _CC)],
                              osem0 if obuf == 0 else osem1).wait()

    def per_roi(r):
        cvec = coords_v[r, :]
        x = cvec[0]
        y = cvec[1]
        w = cvec[2]
        h = cvec[3]
        r_coords3 = (x, y, h, (w - 1) // 8)
        wbv = bnd_v[w, :]
        hbv = bnd_v[h, :]
        wlo = tuple(wbv[q] for q in range(_P))
        whi = tuple(wbv[8 + q] for q in range(_P))
        hlo = tuple(hbv[q] for q in range(_P))
        hhi = tuple(hbv[8 + q] for q in range(_P))
        r_bounds = (h, wlo, whi, hlo, hhi)
        pltpu.sync_copy(wtab_hbm.at[pl.ds(h, 1)], wh_v)
        pltpu.sync_copy(wtab_hbm.at[pl.ds(w, 1)], ww_v)

        crop_rows(r_coords3, 0, 0, dsem0)

        def per_pair(kp, carry2):
            ci0 = 2 * kp
            ci1 = ci0 + 1
            drain_rows(r_coords3, ci0, 0, dsem0)
            crop_rows(r_coords3, ci1, 1, dsem1)

            @pl.when(kp >= 1)
            def _():
                drain_out(r, ci0 - 2, 0)

            compute_chunk(r_bounds, r, ci0, 0, 0)

            drain_rows(r_coords3, ci1, 1, dsem1)

            @pl.when(kp + 1 < _NCK // 2)
            def _():
                crop_rows(r_coords3, ci0 + 2, 0, dsem0)

            @pl.when(kp >= 1)
            def _():
                drain_out(r, ci1 - 2, 1)

            compute_chunk(r_bounds, r, ci1, 1, 1)
            return carry2

        lax.fori_loop(0, _NCK // 2, per_pair, 0)
        drain_out(r, _NCK - 2, 0)
        drain_out(r, _NCK - 1, 1)

    def work_body(k, carry):
        n = plsc.fetch_and_add(cnt_s.at[0], 1, subcore_id=0)

        @pl.when(n < half)
        def _():
            per_roi(cid * half + n)

        return carry

    lax.fori_loop(0, 15, work_body, 0)


@jax.jit
def kernel(feature_map, rois):
    fm = feature_map.reshape(128, 128, _NCH)
    coords = rois.reshape(_NR, 4).astype(jnp.int32)
    coords = jnp.pad(coords, ((0, 0), (0, 12)))
    wtab = jnp.asarray(_WTAB)
    wbnd = jnp.asarray(_WBND)

    call = functools.partial(
        pl.kernel,
        out_type=jax.ShapeDtypeStruct((_NR, _P, _P, _NCH), jnp.float32),
        mesh=plsc.VectorSubcoreMesh(core_axis_name="c", subcore_axis_name="s"),
        compiler_params=pltpu.CompilerParams(use_tc_tiling_on_sc=False),
        scratch_types=[
            pltpu.VMEM((_NR, 16), jnp.int32),
            pltpu.VMEM((1, _M, _P, 16), jnp.float32),
            pltpu.VMEM((1, _M, _P, 16), jnp.float32),
            pltpu.VMEM((32, 16), jnp.int32),
            pltpu.VMEM((2, _M + 1, _M, _CC), jnp.float32),
            pltpu.VMEM((_M + 1, _P, _CC), jnp.float32),
            pltpu.VMEM((2, _P, _P, _CC), jnp.float32),
            pltpu.SMEM((1,), jnp.int32),
            pltpu.SemaphoreType.DMA,
            pltpu.SemaphoreType.DMA,
            pltpu.SemaphoreType.DMA,
            pltpu.SemaphoreType.DMA,
        ],
    )(_roi_body)

    out = call(fm, coords, wtab, wbnd)
    return out.reshape(1, _NR, _P, _P, _NCH)


# speedup vs baseline: 1.0471x; 1.0471x over previous
import functools

import numpy as np
import jax
import jax.numpy as jnp
from jax import lax
from jax.experimental import pallas as pl
from jax.experimental.pallas import tpu as pltpu
from jax.experimental.pallas import tpu_sc as plsc

_P = 7
_M = 31
_NR = 300
_NCH = 512
_CC = 32
_NCK = _NCH // _CC


def _weight_table_np() -> np.ndarray:
    eps = np.float32(np.finfo(np.float32).eps)
    tab = np.zeros((32, _M, _P), np.float32)
    for s in range(1, 32):
        in_size = np.float32(s)
        inv_scale = in_size / np.float32(_P)
        kernel_scale = np.maximum(inv_scale, np.float32(1.0))
        sample_f = ((np.arange(_P, dtype=np.float32) + np.float32(0.5))
                    * inv_scale - np.float32(0.5))
        idx = np.arange(_M, dtype=np.float32)
        xk = np.abs(sample_f[None, :] - idx[:, None]) / kernel_scale
        w = np.maximum(np.float32(0.0), np.float32(1.0) - np.abs(xk))
        w = np.where((idx < in_size)[:, None], w, np.float32(0.0))
        total = np.sum(w, axis=0, keepdims=True)
        w = np.where(np.abs(total) > np.float32(1000.0) * eps,
                     w / np.where(total != 0, total, np.float32(1.0)),
                     np.float32(0.0))
        valid = (sample_f >= np.float32(-0.5)) & (sample_f <= in_size - np.float32(0.5))
        tab[s] = (w * valid[None, :]).astype(np.float32)
    return np.ascontiguousarray(
        np.broadcast_to(tab[:, :, :, None], (32, _M, _P, 16)).astype(np.float32))


_WTAB = _weight_table_np()


def _roi_body(fm_hbm, rois_hbm, wtab_hbm, out_hbm,
              coords_v, wh_v, ww_v, crop_v, tmp_v, outb_v, cnt_s,
              dsem0, dsem1, osem0, osem1, wsem):
    sid = lax.axis_index("s")
    cid = lax.axis_index("c")
    pltpu.sync_copy(rois_hbm, coords_v)
    half = _NR // 2

    @pl.when(sid == 0)
    def _():
        cnt_s[0] = 0

    plsc.subcore_barrier()

    zero = jnp.zeros((16,), jnp.float32)

    _BUCKETS = (8, 16, 24, 31)

    def crop_rows(r_coords, ci, buf, dsem):
        x, y, h, wb_idx = r_coords
        c0 = ci * _CC
        for bj, wb in enumerate(_BUCKETS):
            @pl.when(wb_idx == bj)
            def _(wb=wb):
                def issue(i, carry):
                    pltpu.async_copy(
                        fm_hbm.at[pl.ds(y + i, 1), pl.ds(x, wb), pl.ds(c0, _CC)],
                        crop_v.at[buf, pl.ds(i, 1), pl.ds(0, wb)], dsem)
                    return carry

                lax.fori_loop(0, h, issue, 0)

    def drain_rows(r_coords, ci, buf, dsem):
        x, y, h, wb_idx = r_coords
        c0 = ci * _CC
        for bj, wb in enumerate(_BUCKETS):
            @pl.when(wb_idx == bj)
            def _(wb=wb):
                def drain(i, carry):
                    pltpu.make_async_copy(
                        fm_hbm.at[pl.ds(y + i, 1), pl.ds(x, wb), pl.ds(c0, _CC)],
                        crop_v.at[buf, pl.ds(i, 1), pl.ds(0, wb)], dsem).wait()
                    return carry

                lax.fori_loop(0, h, drain, 0)

    def compute_chunk(r_coords, r, ci, buf, obuf):
        x, y, w, h = r_coords
        c0 = ci * _CC

        def row_a(ib, carry3):
            i0 = 2 * ib
            i1 = i0 + 1

            def col_a(j, accs):
                a0 = crop_v[buf, i0, j, pl.ds(0, 16)]
                a1 = crop_v[buf, i0, j, pl.ds(16, 16)]
                b0 = crop_v[buf, i1, j, pl.ds(0, 16)]
                b1 = crop_v[buf, i1, j, pl.ds(16, 16)]
                new = []
                for q in range(_P):
                    wwv = ww_v[0, j, q, :]
                    new.append(accs[4 * q] + a0 * wwv)
                    new.append(accs[4 * q + 1] + a1 * wwv)
                    new.append(accs[4 * q + 2] + b0 * wwv)
                    new.append(accs[4 * q + 3] + b1 * wwv)
                return tuple(new)

            accs = lax.fori_loop(0, w, col_a, (zero,) * (4 * _P))
            for q in range(_P):
                tmp_v[i0, q, pl.ds(0, 16)] = accs[4 * q]
                tmp_v[i0, q, pl.ds(16, 16)] = accs[4 * q + 1]
                tmp_v[i1, q, pl.ds(0, 16)] = accs[4 * q + 2]
                tmp_v[i1, q, pl.ds(16, 16)] = accs[4 * q + 3]
            return carry3

        lax.fori_loop(0, (h + 1) // 2, row_a, 0)

        for q0 in range(0, _P, 2):
            qs = (q0,) if q0 == _P - 1 else (q0, q0 + 1)

            def row_b(i, accs, qs=qs):
                ts = []
                for q in qs:
                    ts.append(tmp_v[i, q, pl.ds(0, 16)])
                    ts.append(tmp_v[i, q, pl.ds(16, 16)])
                new = list(accs)
                for p in range(_P):
                    whv = wh_v[0, i, p, :]
                    for k in range(2 * len(qs)):
                        new[2 * len(qs) * p + k] = accs[2 * len(qs) * p + k] + whv * ts[k]
                return tuple(new)

            accs = lax.fori_loop(0, h, row_b, (zero,) * (2 * len(qs) * _P))
            for p in range(_P):
                for qi, q in enumerate(qs):
                    outb_v[obuf, p, q, pl.ds(0, 16)] = accs[2 * len(qs) * p + 2 * qi]
                    outb_v[obuf, p, q, pl.ds(16, 16)] = accs[2 * len(qs) * p + 2 * qi + 1]

        pltpu.async_copy(outb_v.at[obuf],
                         out_hbm.at[r, :, :, pl.ds(c0, _CC)],
                         osem0 if obuf == 0 else osem1)

    def drain_out(r, ci, obuf):
        c0 = ci * _CC
        pltpu.make_async_copy(outb_v.at[obuf],
                              out_hbm.at[r, :, :, pl.ds(c0, _CC)],
                              osem0 if obuf == 0 else osem1).wait()

    def per_roi(r):
        cvec = coords_v[r, :]
        x = cvec[0]
        y = cvec[1]
        w = cvec[2]
        h = cvec[3]
        r_coords3 = (x, y, h, (w - 1) // 8)
        r_coords4 = (x, y, w, h)
        wcp_h = pltpu.async_copy(wtab_hbm.at[pl.ds(h, 1)], wh_v, wsem)
        wcp_w = pltpu.async_copy(wtab_hbm.at[pl.ds(w, 1)], ww_v, wsem)

        crop_rows(r_coords3, 0, 0, dsem0)
        wcp_h.wait()
        wcp_w.wait()

        def per_pair(kp, carry2):
            ci0 = 2 * kp
            ci1 = ci0 + 1
            drain_rows(r_coords3, ci0, 0, dsem0)
            crop_rows(r_coords3, ci1, 1, dsem1)

            @pl.when(kp >= 1)
            def _():
                drain_out(r, ci0 - 2, 0)

            compute_chunk(r_coords4, r, ci0, 0, 0)

            drain_rows(r_coords3, ci1, 1, dsem1)

            @pl.when(kp + 1 < _NCK // 2)
            def _():
                crop_rows(r_coords3, ci0 + 2, 0, dsem0)

            @pl.when(kp >= 1)
            def _():
                drain_out(r, ci1 - 2, 1)

            compute_chunk(r_coords4, r, ci1, 1, 1)
            return carry2

        lax.fori_loop(0, _NCK // 2, per_pair, 0)
        drain_out(r, _NCK - 2, 0)
        drain_out(r, _NCK - 1, 1)

    def work_body(k, carry):
        n = plsc.fetch_and_add(cnt_s.at[0], 1, subcore_id=0)

        @pl.when(n < half)
        def _():
            per_roi(cid * half + n)

        return carry

    lax.fori_loop(0, 15, work_body, 0)


@jax.jit
def kernel(feature_map, rois):
    fm = feature_map.reshape(128, 128, _NCH)
    coords = rois.reshape(_NR, 4).astype(jnp.int32)
    coords = jnp.pad(coords, ((0, 0), (0, 12)))
    wtab = jnp.asarray(_WTAB)

    call = functools.partial(
        pl.kernel,
        out_type=jax.ShapeDtypeStruct((_NR, _P, _P, _NCH), jnp.float32),
        mesh=plsc.VectorSubcoreMesh(core_axis_name="c", subcore_axis_name="s"),
        compiler_params=pltpu.CompilerParams(use_tc_tiling_on_sc=False),
        scratch_types=[
            pltpu.VMEM((_NR, 16), jnp.int32),
            pltpu.VMEM((1, _M, _P, 16), jnp.float32),
            pltpu.VMEM((1, _M, _P, 16), jnp.float32),
            pltpu.VMEM((2, _M + 1, _M, _CC), jnp.float32),
            pltpu.VMEM((_M + 1, _P, _CC), jnp.float32),
            pltpu.VMEM((2, _P, _P, _CC), jnp.float32),
            pltpu.SMEM((1,), jnp.int32),
            pltpu.SemaphoreType.DMA,
            pltpu.SemaphoreType.DMA,
            pltpu.SemaphoreType.DMA,
            pltpu.SemaphoreType.DMA,
            pltpu.SemaphoreType.DMA,
        ],
    )(_roi_body)

    out = call(fm, coords, wtab)
    return out.reshape(1, _NR, _P, _P, _NCH)
